# baseline (speedup 1.0000x reference)
import functools

import jax
import jax.numpy as jnp
from jax import lax
from jax.experimental import pallas as pl
from jax.experimental.pallas import tpu as pltpu
from jax.experimental.pallas import tpu_sc as plsc

_BATCH = 16384
_DIM = 64
_NC = 2
_NS = 16
_NW = _NC * _NS
_BPW = _BATCH // _NW
_L = 16
_G = _BPW // _L
_TROW = 8
_GT = 1000000 // _TROW
_SG = 128
_NSG = _BPW // _SG
_SUBG = _SG // _L

_mesh = plsc.VectorSubcoreMesh(core_axis_name="c", subcore_axis_name="s")


@functools.partial(
    pl.kernel,
    mesh=_mesh,
    out_type=jax.ShapeDtypeStruct((_NW, _L), jnp.float32),
    compiler_params=pltpu.CompilerParams(needs_layout_passes=False),
    scratch_types=[
        pltpu.VMEM((_BPW,), jnp.int32),
        pltpu.VMEM((_BPW,), jnp.int32),
        pltpu.VMEM((_BPW,), jnp.float32),
        pltpu.VMEM((_BPW,), jnp.float32),
        pltpu.VMEM((_SG, 1, _DIM), jnp.float32),
        pltpu.VMEM((_SG, 1, _DIM), jnp.float32),
        pltpu.VMEM((_L,), jnp.float32),
        pltpu.SemaphoreType.DMA,
        pltpu.SemaphoreType.DMA,
    ],
)
def _mf_loss_parts(users_hbm, items_hbm, scores_hbm, weights_hbm,
                   utab_hbm, itab_hbm, out_hbm,
                   uidx_v, iidx_v, sc_v, w_v, ubuf_v, ibuf_v, part_v,
                   usem, isem):
    wid = lax.axis_index("s") * _NC + lax.axis_index("c")
    base = wid * _BPW

    pltpu.sync_copy(users_hbm.at[pl.ds(base, _BPW)], uidx_v)
    pltpu.sync_copy(items_hbm.at[pl.ds(base, _BPW)], iidx_v)
    pltpu.sync_copy(scores_hbm.at[pl.ds(base, _BPW)], sc_v)
    pltpu.sync_copy(weights_hbm.at[pl.ds(base, _BPW)], w_v)

    lanes = lax.iota(jnp.int32, _L)
    mask15 = lanes == (_L - 1)

    def sg_body(sg, part):
        sgbase = sg * _SG
        copies = []
        for sub in range(_SUBG):
            gsl = pl.ds(sgbase + sub * _L, _L)
            uvec = uidx_v[gsl]
            ivec = iidx_v[gsl]
            ugrp = uvec >> 3
            igrp = ivec >> 3
            usub = uvec & 7
            isub = ivec & 7
            for j in range(_L):
                p = sub * _L + j
                copies.append(pltpu.async_copy(
                    utab_hbm.at[pl.ds(ugrp[j], 1), pl.ds(usub[j], 1)],
                    ubuf_v.at[pl.ds(p, 1), pl.ds(0, 1)], usem))
                copies.append(pltpu.async_copy(
                    itab_hbm.at[pl.ds(igrp[j], 1), pl.ds(isub[j], 1)],
                    ibuf_v.at[pl.ds(p, 1), pl.ds(0, 1)], isem))
        for c in copies:
            c.wait()

        for sub in range(_SUBG):
            gsl = pl.ds(sgbase + sub * _L, _L)
            s_chunk = sc_v[gsl]
            w_chunk = w_v[gsl]
            for j in range(_L):
                p = sub * _L + j
                prod = jnp.zeros((_L,), jnp.float32)
                for c in range(_DIM // _L):
                    u = ubuf_v[p, 0, pl.ds(c * _L, _L)]
                    v = ibuf_v[p, 0, pl.ds(c * _L, _L)]
                    prod = prod + u * v
                cs = lax.cumsum(prod, axis=0)
                diff = cs - s_chunk[j]
                part = part + diff * diff * w_chunk[j]
        return part

    part = lax.fori_loop(0, _NSG, sg_body, jnp.zeros((_L,), jnp.float32))
    part_v[...] = jnp.where(mask15, part, 0.0)
    pltpu.sync_copy(part_v, out_hbm.at[wid])


def kernel(users, items, scores, sample_weight, user_table, item_table):
    ut3 = user_table.reshape(_GT, _TROW, _DIM)
    it3 = item_table.reshape(_GT, _TROW, _DIM)
    parts = _mf_loss_parts(users, items, scores, sample_weight, ut3, it3)
    return jnp.sum(parts) / _BATCH


# speedup vs baseline: 1.0222x; 1.0222x over previous
import functools

import jax
import jax.numpy as jnp
from jax import lax
from jax.experimental import pallas as pl
from jax.experimental.pallas import tpu as pltpu
from jax.experimental.pallas import tpu_sc as plsc

_BATCH = 16384
_DIM = 64
_NC = 2
_NS = 16
_NW = _NC * _NS
_BPW = _BATCH // _NW
_L = 16
_G = _BPW // _L
_TROW = 8
_GT = 1000000 // _TROW
_SG = 128
_NSG = _BPW // _SG
_SUBG = _SG // _L

_mesh = plsc.VectorSubcoreMesh(core_axis_name="c", subcore_axis_name="s")


@functools.partial(
    pl.kernel,
    mesh=_mesh,
    out_type=jax.ShapeDtypeStruct((_NW, _L), jnp.float32),
    compiler_params=pltpu.CompilerParams(needs_layout_passes=False),
    scratch_types=[
        pltpu.VMEM((_BPW,), jnp.int32),
        pltpu.VMEM((_BPW,), jnp.int32),
        pltpu.VMEM((_BPW,), jnp.float32),
        pltpu.VMEM((_BPW,), jnp.float32),
        pltpu.VMEM((_L, 1, _DIM), jnp.float32),
        pltpu.VMEM((_L, 1, _DIM), jnp.float32),
        pltpu.VMEM((_L,), jnp.float32),
        pltpu.SemaphoreType.DMA,
        pltpu.SemaphoreType.DMA,
    ],
)
def _mf_loss_parts(users_hbm, items_hbm, scores_hbm, weights_hbm,
                   utab_hbm, itab_hbm, out_hbm,
                   uidx_v, iidx_v, sc_v, w_v, ubuf_v, ibuf_v, part_v,
                   usem, isem):
    wid = lax.axis_index("s") * _NC + lax.axis_index("c")
    base = wid * _BPW

    in_copies = [
        pltpu.async_copy(users_hbm.at[pl.ds(base, _BPW)], uidx_v, usem),
        pltpu.async_copy(items_hbm.at[pl.ds(base, _BPW)], iidx_v, isem),
        pltpu.async_copy(scores_hbm.at[pl.ds(base, _BPW)], sc_v, usem),
        pltpu.async_copy(weights_hbm.at[pl.ds(base, _BPW)], w_v, isem),
    ]
    for c in in_copies:
        c.wait()

    lanes = lax.iota(jnp.int32, _L)
    mask15 = lanes == (_L - 1)

    def group_body(g, part):
        gsl = pl.ds(g * _L, _L)
        uvec = uidx_v[gsl]
        ivec = iidx_v[gsl]
        ugrp = uvec >> 3
        igrp = ivec >> 3
        usub = uvec & 7
        isub = ivec & 7
        copies = []
        for j in range(_L):
            copies.append(pltpu.async_copy(
                utab_hbm.at[pl.ds(ugrp[j], 1), pl.ds(usub[j], 1)],
                ubuf_v.at[pl.ds(j, 1), pl.ds(0, 1)], usem))
            copies.append(pltpu.async_copy(
                itab_hbm.at[pl.ds(igrp[j], 1), pl.ds(isub[j], 1)],
                ibuf_v.at[pl.ds(j, 1), pl.ds(0, 1)], isem))
        for c in copies:
            c.wait()

        s_chunk = sc_v[gsl]
        w_chunk = w_v[gsl]
        for j in range(_L):
            prod = jnp.zeros((_L,), jnp.float32)
            for c in range(_DIM // _L):
                u = ubuf_v[j, 0, pl.ds(c * _L, _L)]
                v = ibuf_v[j, 0, pl.ds(c * _L, _L)]
                prod = prod + u * v
            cs = lax.cumsum(prod, axis=0)
            diff = cs - s_chunk[j]
            part = part + diff * diff * w_chunk[j]
        return part

    part = lax.fori_loop(0, _G, group_body, jnp.zeros((_L,), jnp.float32))
    part_v[...] = jnp.where(mask15, part, 0.0)
    pltpu.sync_copy(part_v, out_hbm.at[wid])


def kernel(users, items, scores, sample_weight, user_table, item_table):
    ut3 = user_table.reshape(_GT, _TROW, _DIM)
    it3 = item_table.reshape(_GT, _TROW, _DIM)
    parts = _mf_loss_parts(users, items, scores, sample_weight, ut3, it3)
    return jnp.sum(parts) / _BATCH
